# baseline (speedup 1.0000x reference)
import functools

import jax
import jax.numpy as jnp
from jax import lax
from jax.experimental import pallas as pl
from jax.experimental.pallas import tpu as pltpu
from jax.experimental.pallas import tpu_sc as plsc

_NC = 2
_NS = 16
_NL = 16
_NW = _NC * _NS
_C = 128


def _enc_body(xn_ref, xh_ref, wn_ref, wh_ref, on_ref, oh_ref):
    on_ref[...] = jnp.dot(xn_ref[...], wn_ref[...],
                          preferred_element_type=jnp.float32)
    oh_ref[...] = jnp.dot(xh_ref[...], wh_ref[...],
                          preferred_element_type=jnp.float32)


def _encode(x_node_feat, x_he_feat, W_node, W_he):
    N, DF = x_node_feat.shape
    DH = W_node.shape[1]
    BR = 1000
    return pl.pallas_call(
        _enc_body,
        grid=(N // BR,),
        in_specs=[
            pl.BlockSpec((BR, DF), lambda i: (i, 0)),
            pl.BlockSpec((BR, DF), lambda i: (i, 0)),
            pl.BlockSpec((DF, DH), lambda i: (0, 0)),
            pl.BlockSpec((DF, DH), lambda i: (0, 0)),
        ],
        out_specs=[
            pl.BlockSpec((BR, DH), lambda i: (i, 0)),
            pl.BlockSpec((BR, DH), lambda i: (i, 0)),
        ],
        out_shape=[
            jax.ShapeDtypeStruct((N, DH), jnp.float32),
            jax.ShapeDtypeStruct((N, DH), jnp.float32),
        ],
    )(x_node_feat, x_he_feat, W_node, W_he)


def _gum_body(n_valid, u0_ref, u1_ref, bd_ref, thr_ref):
    t = (jnp.log(-jnp.log(u1_ref[...])) - jnp.log(-jnp.log(u0_ref[...]))
         - bd_ref[...])
    R, Ccol = thr_ref.shape
    flat = (lax.broadcasted_iota(jnp.int32, (R, Ccol), 0) * Ccol
            + lax.broadcasted_iota(jnp.int32, (R, Ccol), 1))
    thr_ref[...] = jnp.where(flat < n_valid, t, jnp.inf)


def _gumbel_thr(u0, u1, bd_row, n_valid):
    R, Ccol = u0.shape
    return pl.pallas_call(
        functools.partial(_gum_body, n_valid),
        out_shape=jax.ShapeDtypeStruct((R, Ccol), jnp.float32),
    )(u0, u1, bd_row)


def _rne_bf16(x):
    b = plsc.bitcast(x, jnp.uint32)
    lsb = (b >> jnp.uint32(16)) & jnp.uint32(1)
    r = (b + jnp.uint32(0x7FFF) + lsb) & jnp.uint32(0xFFFF0000)
    return plsc.bitcast(r, jnp.float32)


def _decode_sc(xn, xhd, wdb, src_p, dst_p, thr_p):
    e_pad = thr_p.shape[0]
    DH = xn.shape[1]
    per_w = e_pad // _NW
    nchunk = per_w // _C
    ngroup = _C // _NL
    mesh = plsc.VectorSubcoreMesh(core_axis_name="c", subcore_axis_name="s")

    @functools.partial(
        pl.kernel,
        mesh=mesh,
        out_type=[
            jax.ShapeDtypeStruct((e_pad,), jnp.float32),
            jax.ShapeDtypeStruct((_NW, _NL), jnp.float32),
        ],
        scratch_types=[
            pltpu.VMEM((nchunk, _C), jnp.int32),
            pltpu.VMEM((nchunk, _C), jnp.int32),
            pltpu.VMEM((per_w,), jnp.float32),
            pltpu.VMEM((per_w,), jnp.float32),
            pltpu.VMEM((2, _C, DH), jnp.float32),
            pltpu.VMEM((2, _C, DH), jnp.float32),
            pltpu.VMEM((DH,), jnp.float32),
            pltpu.VMEM((_NL,), jnp.float32),
            pltpu.SemaphoreType.DMA,
            pltpu.SemaphoreType.DMA,
            pltpu.SemaphoreType.DMA,
            pltpu.SemaphoreType.DMA,
        ],
        compiler_params=pltpu.CompilerParams(needs_layout_passes=False),
    )
    def k(xn_hbm, xh_hbm, wd_hbm, src_hbm, dst_hbm, thr_hbm, keep_hbm, cnt_hbm,
          srcv, dstv, thrv, keepv, av, bv, wdv, cntv, semA0, semB0, semA1,
          semB1):
        wid = lax.axis_index("s") * _NC + lax.axis_index("c")
        base_w = wid * per_w
        iota16 = lax.iota(jnp.int32, _NL)
        sems = ((semA0, semB0), (semA1, semB1))
        c_offs = [(iota16 + j) & (_NL - 1) for j in range(_NL)]

        pltpu.sync_copy(wd_hbm, wdv)
        pltpu.sync_copy(src_hbm.at[wid], srcv)
        pltpu.sync_copy(dst_hbm.at[wid], dstv)
        pltpu.sync_copy(thr_hbm.at[pl.ds(base_w, per_w)], thrv)

        def start(ci, slot):
            pltpu.async_copy(xn_hbm.at[srcv.at[ci]],
                             av.at[slot], sems[slot][0])
            pltpu.async_copy(xh_hbm.at[dstv.at[ci]],
                             bv.at[slot], sems[slot][1])

        def wait(ci, slot):
            pltpu.make_async_copy(xn_hbm.at[srcv.at[ci]],
                                  av.at[slot], sems[slot][0]).wait()
            pltpu.make_async_copy(xh_hbm.at[dstv.at[ci]],
                                  bv.at[slot], sems[slot][1]).wait()

        def compute(ci, slot):
            a2d = av.at[slot]
            b2d = bv.at[slot]

            def group_body(g, _):
                rows = g * _NL + iota16

                accs = [jnp.zeros((_NL,), jnp.float32) for _ in range(8)]
                for k in range(DH):
                    s, j = divmod(k, _NL)
                    c = c_offs[j] + (s * _NL)
                    p = _rne_bf16(plsc.load_gather(a2d, [rows, c])
                                  * plsc.load_gather(b2d, [rows, c]))
                    w = plsc.load_gather(wdv, [c])
                    accs[k % 8] = accs[k % 8] + p * w
                acc = (((accs[0] + accs[1]) + (accs[2] + accs[3]))
                       + ((accs[4] + accs[5]) + (accs[6] + accs[7])))
                off = ci * _C + g * _NL
                thrg = thrv[pl.ds(off, _NL)]
                keep = jnp.where(acc > thrg, 1.0, 0.0).astype(jnp.float32)
                keepv[pl.ds(off, _NL)] = keep
                return 0

            lax.fori_loop(0, ngroup, group_body, 0)

        start(0, 0)
        start(1, 1)

        def outer_body(po, _):
            for b in range(2):
                ci = po * 2 + b
                wait(ci, b)
                compute(ci, b)

                @pl.when(ci + 2 < nchunk)
                def _():
                    start(ci + 2, b)
            return 0

        lax.fori_loop(0, nchunk // 2, outer_body, 0)

        def cnt_body(i, acc):
            return acc + keepv[pl.ds(i * _NL, _NL)]

        cnt = lax.fori_loop(0, per_w // _NL, cnt_body,
                            jnp.zeros((_NL,), jnp.float32))
        cntv[...] = cnt
        pltpu.sync_copy(keepv, keep_hbm.at[pl.ds(base_w, per_w)])
        pltpu.sync_copy(cntv, cnt_hbm.at[wid])

    return k(xn, xhd, wdb, src_p, dst_p, thr_p)


def kernel(x_node_feat, x_he_feat, W_node, W_he, W_dec, b_dec, edge_index,
           num_ori_edge, gumbel_u):
    n_ori = gumbel_u.shape[0]
    n_edges = edge_index.shape[1]
    DH = W_node.shape[1]
    blk = _NW * _C
    e_pad = ((n_ori + blk - 1) // blk) * blk

    wdb = (W_dec[:, 1].astype(jnp.bfloat16).astype(jnp.float32)
           - W_dec[:, 0].astype(jnp.bfloat16).astype(jnp.float32))
    bd = b_dec[1] - b_dec[0]
    bd_row = jnp.full((1, _C), bd, jnp.float32)

    zero_dep = jnp.asarray(num_ori_edge, dtype=edge_index.dtype) - n_ori
    src_p = jnp.pad(edge_index[0, :n_ori] + zero_dep,
                    (0, e_pad - n_ori)).astype(jnp.int32)
    dst_p = jnp.pad(edge_index[1, :n_ori] + zero_dep,
                    (0, e_pad - n_ori)).astype(jnp.int32)

    gup = jnp.pad(gumbel_u, ((0, e_pad - n_ori), (0, 0)), constant_values=0.5)
    R = e_pad // _C
    u0 = gup[:, 0].reshape(R, _C)
    u1 = gup[:, 1].reshape(R, _C)

    xn, xhd = _encode(x_node_feat, x_he_feat, W_node, W_he)
    thr_p = _gumbel_thr(u0, u1, bd_row, n_ori).reshape(e_pad)

    nchunk = e_pad // (_NW * _C)
    src_3d = src_p.reshape(_NW, nchunk, _C)
    dst_3d = dst_p.reshape(_NW, nchunk, _C)
    keep_p, counts = _decode_sc(xn, xhd, wdb, src_3d, dst_3d, thr_p)

    keep = keep_p[:n_ori]
    deg = 1.0 - jnp.sum(counts) / jnp.float32(n_ori)
    full = jnp.concatenate(
        [keep, jnp.ones((n_edges - n_ori,), jnp.float32)], axis=0)
    return (full, deg)


# speedup vs baseline: 1.2484x; 1.2484x over previous
import functools

import jax
import jax.numpy as jnp
from jax import lax
from jax.experimental import pallas as pl
from jax.experimental.pallas import tpu as pltpu
from jax.experimental.pallas import tpu_sc as plsc

_NC = 2
_NS = 16
_NL = 16
_NW = _NC * _NS
_C = 64
_NB = 4


def _enc_body(xn_ref, xh_ref, wn_ref, wh_ref, on_ref, oh_ref):
    on_ref[...] = jnp.dot(xn_ref[...], wn_ref[...],
                          preferred_element_type=jnp.float32)
    oh_ref[...] = jnp.dot(xh_ref[...], wh_ref[...],
                          preferred_element_type=jnp.float32)


def _encode(x_node_feat, x_he_feat, W_node, W_he):
    N, DF = x_node_feat.shape
    DH = W_node.shape[1]
    BR = 1000
    return pl.pallas_call(
        _enc_body,
        grid=(N // BR,),
        in_specs=[
            pl.BlockSpec((BR, DF), lambda i: (i, 0)),
            pl.BlockSpec((BR, DF), lambda i: (i, 0)),
            pl.BlockSpec((DF, DH), lambda i: (0, 0)),
            pl.BlockSpec((DF, DH), lambda i: (0, 0)),
        ],
        out_specs=[
            pl.BlockSpec((BR, DH), lambda i: (i, 0)),
            pl.BlockSpec((BR, DH), lambda i: (i, 0)),
        ],
        out_shape=[
            jax.ShapeDtypeStruct((N, DH), jnp.float32),
            jax.ShapeDtypeStruct((N, DH), jnp.float32),
        ],
    )(x_node_feat, x_he_feat, W_node, W_he)


def _gum_body(n_valid, u0_ref, u1_ref, bd_ref, thr_ref):
    t = (jnp.log(-jnp.log(u1_ref[...])) - jnp.log(-jnp.log(u0_ref[...]))
         - bd_ref[...])
    R, Ccol = thr_ref.shape
    flat = (lax.broadcasted_iota(jnp.int32, (R, Ccol), 0) * Ccol
            + lax.broadcasted_iota(jnp.int32, (R, Ccol), 1))
    thr_ref[...] = jnp.where(flat < n_valid, t, jnp.inf)


def _gumbel_thr(u0, u1, bd_row, n_valid):
    R, Ccol = u0.shape
    return pl.pallas_call(
        functools.partial(_gum_body, n_valid),
        out_shape=jax.ShapeDtypeStruct((R, Ccol), jnp.float32),
    )(u0, u1, bd_row)


def _rne_bf16(x):
    b = plsc.bitcast(x, jnp.uint32)
    lsb = (b >> jnp.uint32(16)) & jnp.uint32(1)
    r = (b + jnp.uint32(0x7FFF) + lsb) & jnp.uint32(0xFFFF0000)
    return plsc.bitcast(r, jnp.float32)


def _decode_sc(xn, xhd, wdb, src_p, dst_p, thr_p):
    e_pad = thr_p.shape[0]
    DH = xn.shape[1]
    per_w = e_pad // _NW
    nchunk = per_w // _C
    ngroup = _C // _NL
    mesh = plsc.VectorSubcoreMesh(core_axis_name="c", subcore_axis_name="s")

    @functools.partial(
        pl.kernel,
        mesh=mesh,
        out_type=[
            jax.ShapeDtypeStruct((e_pad,), jnp.float32),
            jax.ShapeDtypeStruct((_NW, _NL), jnp.float32),
        ],
        scratch_types=[
            pltpu.VMEM((nchunk, _C), jnp.int32),
            pltpu.VMEM((nchunk, _C), jnp.int32),
            pltpu.VMEM((per_w,), jnp.float32),
            pltpu.VMEM((per_w,), jnp.float32),
            pltpu.VMEM((_NB, _C, DH), jnp.float32),
            pltpu.VMEM((_NB, _C, DH), jnp.float32),
            pltpu.VMEM((DH,), jnp.float32),
            pltpu.VMEM((_NL,), jnp.float32),
        ] + [pltpu.SemaphoreType.DMA] * (2 * _NB),
        compiler_params=pltpu.CompilerParams(needs_layout_passes=False),
    )
    def k(xn_hbm, xh_hbm, wd_hbm, src_hbm, dst_hbm, thr_hbm, keep_hbm, cnt_hbm,
          srcv, dstv, thrv, keepv, av, bv, wdv, cntv, *allsems):
        wid = lax.axis_index("s") * _NC + lax.axis_index("c")
        base_w = wid * per_w
        iota16 = lax.iota(jnp.int32, _NL)
        sems = tuple((allsems[2 * i], allsems[2 * i + 1]) for i in range(_NB))
        c_offs = [(iota16 + j) & (_NL - 1) for j in range(_NL)]

        pltpu.sync_copy(wd_hbm, wdv)
        pltpu.sync_copy(src_hbm.at[wid], srcv)
        pltpu.sync_copy(dst_hbm.at[wid], dstv)
        pltpu.sync_copy(thr_hbm.at[pl.ds(base_w, per_w)], thrv)

        def start(ci, slot):
            pltpu.async_copy(xn_hbm.at[srcv.at[ci]],
                             av.at[slot], sems[slot][0])
            pltpu.async_copy(xh_hbm.at[dstv.at[ci]],
                             bv.at[slot], sems[slot][1])

        def wait(ci, slot):
            pltpu.make_async_copy(xn_hbm.at[srcv.at[ci]],
                                  av.at[slot], sems[slot][0]).wait()
            pltpu.make_async_copy(xh_hbm.at[dstv.at[ci]],
                                  bv.at[slot], sems[slot][1]).wait()

        def compute(ci, slot):
            a2d = av.at[slot]
            b2d = bv.at[slot]

            def group_body(g, _):
                rows = g * _NL + iota16

                accs = [jnp.zeros((_NL,), jnp.float32) for _ in range(8)]
                for k in range(DH):
                    s, j = divmod(k, _NL)
                    c = c_offs[j] + (s * _NL)
                    p = _rne_bf16(plsc.load_gather(a2d, [rows, c])
                                  * plsc.load_gather(b2d, [rows, c]))
                    w = plsc.load_gather(wdv, [c])
                    accs[k % 8] = accs[k % 8] + p * w
                acc = (((accs[0] + accs[1]) + (accs[2] + accs[3]))
                       + ((accs[4] + accs[5]) + (accs[6] + accs[7])))
                off = ci * _C + g * _NL
                thrg = thrv[pl.ds(off, _NL)]
                keep = jnp.where(acc > thrg, 1.0, 0.0).astype(jnp.float32)
                keepv[pl.ds(off, _NL)] = keep
                return 0

            lax.fori_loop(0, ngroup, group_body, 0)

        for b in range(_NB):
            start(b, b)

        def outer_body(po, _):
            for b in range(_NB):
                ci = po * _NB + b
                wait(ci, b)
                compute(ci, b)

                @pl.when(ci + _NB < nchunk)
                def _():
                    start(ci + _NB, b)
            return 0

        lax.fori_loop(0, nchunk // _NB, outer_body, 0)

        def cnt_body(i, acc):
            return acc + keepv[pl.ds(i * _NL, _NL)]

        cnt = lax.fori_loop(0, per_w // _NL, cnt_body,
                            jnp.zeros((_NL,), jnp.float32))
        cntv[...] = cnt
        pltpu.sync_copy(keepv, keep_hbm.at[pl.ds(base_w, per_w)])
        pltpu.sync_copy(cntv, cnt_hbm.at[wid])

    return k(xn, xhd, wdb, src_p, dst_p, thr_p)


def kernel(x_node_feat, x_he_feat, W_node, W_he, W_dec, b_dec, edge_index,
           num_ori_edge, gumbel_u):
    n_ori = gumbel_u.shape[0]
    n_edges = edge_index.shape[1]
    DH = W_node.shape[1]
    blk = _NW * _C
    e_pad = ((n_ori + blk - 1) // blk) * blk

    wdb = (W_dec[:, 1].astype(jnp.bfloat16).astype(jnp.float32)
           - W_dec[:, 0].astype(jnp.bfloat16).astype(jnp.float32))
    bd = b_dec[1] - b_dec[0]
    bd_row = jnp.full((1, _C), bd, jnp.float32)

    zero_dep = jnp.asarray(num_ori_edge, dtype=edge_index.dtype) - n_ori
    src_p = jnp.pad(edge_index[0, :n_ori] + zero_dep,
                    (0, e_pad - n_ori)).astype(jnp.int32)
    dst_p = jnp.pad(edge_index[1, :n_ori] + zero_dep,
                    (0, e_pad - n_ori)).astype(jnp.int32)

    gup = jnp.pad(gumbel_u, ((0, e_pad - n_ori), (0, 0)), constant_values=0.5)
    R = e_pad // _C
    u0 = gup[:, 0].reshape(R, _C)
    u1 = gup[:, 1].reshape(R, _C)

    xn, xhd = _encode(x_node_feat, x_he_feat, W_node, W_he)
    thr_p = _gumbel_thr(u0, u1, bd_row, n_ori).reshape(e_pad)

    nchunk = e_pad // (_NW * _C)
    src_3d = src_p.reshape(_NW, nchunk, _C)
    dst_3d = dst_p.reshape(_NW, nchunk, _C)
    keep_p, counts = _decode_sc(xn, xhd, wdb, src_3d, dst_3d, thr_p)

    keep = keep_p[:n_ori]
    deg = 1.0 - jnp.sum(counts) / jnp.float32(n_ori)
    full = jnp.concatenate(
        [keep, jnp.ones((n_edges - n_ori,), jnp.float32)], axis=0)
    return (full, deg)
